# baseline (speedup 1.0000x reference)
import jax
import jax.numpy as jnp
from jax.experimental import pallas as pl
from jax.experimental.pallas import tpu as pltpu

BATCH, SIZE_IN, SIZE_OUT = 64, 4096, 1024
BLOCK_I = 4096


def _fused_kernel(x_ref, a_ref, o_ref):
    a = a_ref[...]
    amax = jnp.max(a, axis=1)
    amin = jnp.min(a, axis=1)
    wmax = jax.nn.sigmoid(amax)
    wmin = jax.nn.sigmoid(amin)
    x = x_ref[...]
    o_ref[...] = x * jnp.where(x >= 0.0, wmax[None, :], wmin[None, :])


def kernel(x, A):
    return pl.pallas_call(
        _fused_kernel,
        grid=(SIZE_IN // BLOCK_I,),
        in_specs=[
            pl.BlockSpec((BATCH, BLOCK_I), lambda i: (0, i)),
            pl.BlockSpec((BLOCK_I, SIZE_OUT), lambda i: (i, 0)),
        ],
        out_specs=pl.BlockSpec((BATCH, BLOCK_I), lambda i: (0, i)),
        out_shape=jax.ShapeDtypeStruct((BATCH, SIZE_IN), jnp.float32),
        compiler_params=pltpu.CompilerParams(
            dimension_semantics=("arbitrary",),
            vmem_limit_bytes=56 * 1024 * 1024,
        ),
    )(x, A)


# speedup vs baseline: 1.1197x; 1.1197x over previous
import jax
import jax.numpy as jnp
from jax.experimental import pallas as pl
from jax.experimental.pallas import tpu as pltpu

BATCH, SIZE_IN, SIZE_OUT = 64, 4096, 1024
BLOCK_I = 2048


def _fused_kernel(x_ref, a_ref, o_ref):
    a = a_ref[...]
    amax = jnp.max(a, axis=1)
    amin = jnp.min(a, axis=1)
    wmax = jax.nn.sigmoid(amax)
    wmin = jax.nn.sigmoid(amin)
    x = x_ref[...]
    o_ref[...] = x * jnp.where(x >= 0.0, wmax[None, :], wmin[None, :])


def kernel(x, A):
    return pl.pallas_call(
        _fused_kernel,
        grid=(SIZE_IN // BLOCK_I,),
        in_specs=[
            pl.BlockSpec((BATCH, BLOCK_I), lambda i: (0, i)),
            pl.BlockSpec((BLOCK_I, SIZE_OUT), lambda i: (i, 0)),
        ],
        out_specs=pl.BlockSpec((BATCH, BLOCK_I), lambda i: (0, i)),
        out_shape=jax.ShapeDtypeStruct((BATCH, SIZE_IN), jnp.float32),
        compiler_params=pltpu.CompilerParams(
            dimension_semantics=("parallel",),
        ),
    )(x, A)
